# Initial kernel scaffold; baseline (speedup 1.0000x reference)
import jax
import jax.numpy as jnp
from jax.experimental import pallas as pl


def kernel(x, edge_index, edge_in, in_w, edge_out, out_w, edge_index_tuple, edge_weight_tuple, lin1_w, ib1_ln_w, ib1_ln_b, ib1_c1_w, ib1_c1_b, ib1_c2_w, ib1_c2_b, conv_w, conv_b, bn1_g, bn1_b, bn2_g, bn2_b):
    raise NotImplementedError("write your pallas kernel here")


# speedup vs baseline: 71.2084x; 71.2084x over previous
import functools

import jax
import jax.numpy as jnp
from jax import lax
from jax.experimental import pallas as pl
from jax.experimental.pallas import tpu as pltpu
from jax.experimental.pallas import tpu_sc as plsc

N = 10000
E = 160000
D = 256
OUT_DIM = 128
EPS = 1e-5
BSH = 11

NPAD = 10240
HALF = NPAD // 2
NR = NPAD // 128
NS = 16
SLOTS_W = E // NS
CHUNK = 2000
NGRP = CHUNK // 16
CBUF = 2048
KROW = 32
ROWS_W = HALF // NS
SLICE = NPAD // NS
SRR = SLICE // 128
CAP = 512
SENT = 1 << 30

_f32 = jnp.float32
_i32 = jnp.int32


def _sc_body(e0, e1, e2, e3, symx, xc1, xc2, hacc,
             st, degp, cu, cvl, cnm, slu_v, slw_v,
             idxb, idg, rows, sl_a, sl_b, sl_cf, sl_t2,
             acc, deg0, deg1, deg2, dis0, dis1, dis2, slu, slw, cfb):
  cid = lax.axis_index("c")
  sid = lax.axis_index("s")
  iota16 = lax.iota(_i32, 16)
  fz = jnp.zeros((16,), _f32)
  fone = jnp.ones((16,), _f32)
  iz = jnp.zeros((16,), _i32)

  def rsqrt_nr(x):
    i = plsc.bitcast(x, _i32)
    y = plsc.bitcast(jnp.full((16,), 0x5F3759DF, _i32) - (i >> 1), _f32)
    for _ in range(3):
      y = y * (1.5 - 0.5 * x * y * y)
    return y

  def _zrow(r, c):
    for g in range(16):
      rows[r, pl.ds(g * 16, 16)] = fz
    return c
  lax.fori_loop(0, KROW, _zrow, 0)
  for t in range(ROWS_W // KROW):
    pltpu.sync_copy(rows, acc.at[pl.ds(sid * ROWS_W + t * KROW, KROW)])

  def _zdeg(g, c):
    degp[g >> 3, pl.ds((g & 7) * 16, 16)] = fz
    return c
  lax.fori_loop(0, NPAD // 16, _zdeg, 0)
  for dg in (deg0, deg1, deg2):
    pltpu.sync_copy(degp.at[pl.ds(0, SRR)], dg.at[pl.ds(sid * SRR, SRR)])
  for g in range(5):
    idg[0, pl.ds(g * 16, 16)] = g * 16 + iota16
  plsc.subcore_barrier()

  def msg_pass(eh, tbl, dgcn, use_w):
    for c in range(SLOTS_W // CHUNK):
      off = sid * SLOTS_W + c * CHUNK
      pltpu.sync_copy(eh.at[:, pl.ds(off, CHUNK)], st)

      def zb(g, cc):
        s = pl.ds(g * 16, 16)
        cu[s] = iz
        cvl[s] = iz
        cnm[s] = fz
        return cc
      lax.fori_loop(0, CBUF // 16, zb, 0)

      def cmp(g, cnt):
        s = pl.ds(g * 16, 16)
        u = st[0, s]
        v = st[1, s]
        w = plsc.bitcast(st[2, s], _f32) if use_w else fone
        act = (u >> BSH) == (v >> BSH)
        if dgcn:
          act = act & (u != v)
        m = act & ((v >= cid * HALF) & (v < cid * HALF + HALF))
        t = pl.ds(cnt, 16)
        plsc.store_compressed(cu.at[t], u, mask=m)
        plsc.store_compressed(cvl.at[t], v - cid * HALF, mask=m)
        plsc.store_compressed(cnm.at[t], w, mask=m)
        return cnt + jnp.max(plsc.all_reduce_population_count(m))
      cnt = lax.fori_loop(0, NGRP, cmp, jnp.asarray(0, _i32))

      if dgcn:
        def nrm(g, cc):
          s = pl.ds(g * 16, 16)
          uu = cu[s]
          vv = cvl[s] + cid * HALF
          du = plsc.load_gather(degp, [uu >> 7, uu & 127])
          dv = plsc.load_gather(degp, [vv >> 7, vv & 127])
          lane = (g * 16 + iota16) < cnt
          cnm[s] = jnp.where(lane, du * dv * cnm[s], 0.0)
          return cc
        lax.fori_loop(0, (cnt + 15) // 16, nrm, 0)

      def sub(j, cc):
        b = j * KROW
        pltpu.sync_copy(tbl.at[cu.at[pl.ds(b, KROW)]], rows)

        def scale(e, c2):
          nm = plsc.load_gather(cnm, [jnp.full((16,), 1, _i32) * (b + e)])
          for g in range(16):
            cs = pl.ds(g * 16, 16)
            rows[e, cs] = rows[e, cs] * nm
          return c2
        lax.fori_loop(0, KROW, scale, 0)
        for g in range(KROW // 16):
          idxb[0, pl.ds(g * 16, 16)] = cvl[pl.ds(b + g * 16, 16)]
        pltpu.sync_copy(rows, acc.at[idxb.at[0]], add=True)
        return cc
      lax.fori_loop(0, (cnt + KROW - 1) // KROW, sub, 0)

  msg_pass(e0, xc1, False, True)
  msg_pass(e3, xc2, False, True)

  def deg_scan(eh, dg, use_w, collect_sl):
    def _zd(g, c):
      degp[g >> 3, pl.ds((g & 7) * 16, 16)] = fz
      return c
    lax.fori_loop(0, NPAD // 16, _zd, 0)
    if collect_sl:
      def _zs(g, c):
        slu_v[pl.ds(g * 16, 16)] = iz + SENT
        return c
      lax.fori_loop(0, CAP // 16, _zs, 0)

    def chunk_scan(c, scnt):
      off = sid * SLOTS_W + c * CHUNK
      pltpu.sync_copy(eh.at[:, pl.ds(off, CHUNK)], st)

      def grp(g, sc2):
        s = pl.ds(g * 16, 16)
        u = st[0, s]
        v = st[1, s]
        w = plsc.bitcast(st[2, s], _f32) if use_w else fone
        act = (u >> BSH) == (v >> BSH)
        me = act & (u != v)
        plsc.addupdate_scatter(degp, [u >> 7, u & 127], w, mask=me)
        if collect_sl:
          ms = act & (u == v) & ((jnp.full((16,), 1, _i32) * sc2) < CAP - 15)
          t = pl.ds(sc2, 16)
          plsc.store_compressed(slu_v.at[t], u, mask=ms)
          plsc.store_compressed(slw_v.at[t], w, mask=ms)
          sc2 = sc2 + jnp.max(plsc.all_reduce_population_count(ms))
        return sc2
      return lax.fori_loop(0, NGRP, grp, scnt)
    lax.fori_loop(0, SLOTS_W // CHUNK, chunk_scan, jnp.asarray(0, _i32))
    pltpu.sync_copy(degp, dg.at[idg.at[0]], add=True)

  def merge_sl(target):
    def _init(g, c):
      target[pl.ds(g * 16, 16)] = fone
      return c
    lax.fori_loop(0, SLICE // 16, _init, 0)
    nb = sid * SLICE
    for j in range(NS):
      pltpu.sync_copy(slu.at[j], slu_v)
      pltpu.sync_copy(slw.at[j], slw_v)

      def mg(g, c):
        s = pl.ds(g * 16, 16)
        uu = slu_v[s]
        ww = slw_v[s]
        m = (uu >= nb) & (uu < nb + SLICE)
        li = jnp.where(m, uu - nb, 0)
        plsc.store_scatter(target, [li], ww, mask=m)
        return c
      lax.fori_loop(0, CAP // 16, mg, 0)

  deg_scan(e0, deg0, False, False)
  deg_scan(e1, deg1, True, True)
  pltpu.sync_copy(slu_v, slu.at[sid])
  pltpu.sync_copy(slw_v, slw.at[sid])
  plsc.subcore_barrier()
  merge_sl(sl_a)
  plsc.subcore_barrier()
  deg_scan(e2, deg2, True, True)
  pltpu.sync_copy(slu_v, slu.at[sid])
  pltpu.sync_copy(slw_v, slw.at[sid])
  plsc.subcore_barrier()
  merge_sl(sl_b)
  plsc.subcore_barrier()

  def _zcf(g, c):
    sl_cf[pl.ds(g * 16, 16)] = fz
    return c
  lax.fori_loop(0, SLICE // 16, _zcf, 0)
  for dg, ds_sh, lwsl in ((deg0, dis0, None), (deg1, dis1, sl_a),
                          (deg2, dis2, sl_b)):
    pltpu.sync_copy(dg.at[pl.ds(sid * SRR, SRR)], degp.at[pl.ds(0, SRR)])

    def fin(g, c):
      s = pl.ds((g & 7) * 16, 16)
      fs = pl.ds(g * 16, 16)
      lwv = fone if lwsl is None else lwsl[fs]
      dtot = degp[g >> 3, s] + lwv
      pos = dtot > 0.0
      dsafe = jnp.where(pos, dtot, 1.0)
      sl_cf[fs] = sl_cf[fs] + jnp.where(pos, lwv / dsafe, 0.0)
      sl_t2[g >> 3, s] = jnp.where(pos, rsqrt_nr(dsafe), 0.0)
      return c
    lax.fori_loop(0, SLICE // 16, fin, 0)
    pltpu.sync_copy(sl_t2, ds_sh.at[pl.ds(sid * SRR, SRR)])
  pltpu.sync_copy(sl_cf, cfb.at[pl.ds(sid * SLICE, SLICE)])
  plsc.subcore_barrier()

  pltpu.sync_copy(dis0, degp)
  msg_pass(e0, symx, True, False)
  pltpu.sync_copy(dis1, degp)
  msg_pass(e1, symx, True, True)
  pltpu.sync_copy(dis2, degp)
  msg_pass(e2, symx, True, True)

  rg = cid * HALF + sid * ROWS_W
  rl = sid * ROWS_W
  pltpu.sync_copy(cfb.at[pl.ds(rg, ROWS_W)], sl_cf.at[pl.ds(0, ROWS_W)])
  for t in range(ROWS_W // KROW):
    b = t * KROW
    pltpu.sync_copy(symx.at[pl.ds(rg + b, KROW)], rows)

    def lscale(e, cc):
      nm = plsc.load_gather(sl_cf, [jnp.full((16,), 1, _i32) * (b + e)])
      for g in range(16):
        cs = pl.ds(g * 16, 16)
        rows[e, cs] = rows[e, cs] * nm
      return cc
    lax.fori_loop(0, KROW, lscale, 0)
    for g in range(KROW // 16):
      idxb[0, pl.ds(g * 16, 16)] = (rl + b + g * 16) + iota16
    pltpu.sync_copy(rows, acc.at[idxb.at[0]], add=True)

  plsc.subcore_barrier()
  pltpu.sync_copy(acc.at[pl.ds(sid * ROWS_W, ROWS_W)],
                  hacc.at[pl.ds(cid * HALF + sid * ROWS_W, ROWS_W)])


def _sc_scatter(e0, e1, e2, e3, symx, xc1, xc2):
  mesh = plsc.VectorSubcoreMesh(core_axis_name="c", subcore_axis_name="s")
  return pl.kernel(
      _sc_body,
      out_type=jax.ShapeDtypeStruct((NPAD, D), _f32),
      mesh=mesh,
      compiler_params=pltpu.CompilerParams(use_tc_tiling_on_sc=False,
                                           needs_layout_passes=False),
      scratch_types=[
          pltpu.VMEM((3, CHUNK), _i32),
          pltpu.VMEM((NR, 128), _f32),
          pltpu.VMEM((CBUF,), _i32),
          pltpu.VMEM((CBUF,), _i32),
          pltpu.VMEM((CBUF,), _f32),
          pltpu.VMEM((CAP,), _i32),
          pltpu.VMEM((CAP,), _f32),
          pltpu.VMEM((1, KROW), _i32),
          pltpu.VMEM((1, NR), _i32),
          pltpu.VMEM((KROW, D), _f32),
          pltpu.VMEM((SLICE,), _f32),
          pltpu.VMEM((SLICE,), _f32),
          pltpu.VMEM((SLICE,), _f32),
          pltpu.VMEM((SRR, 128), _f32),
          pltpu.VMEM_SHARED((HALF, D), _f32),
          pltpu.VMEM_SHARED((NR, 128), _f32),
          pltpu.VMEM_SHARED((NR, 128), _f32),
          pltpu.VMEM_SHARED((NR, 128), _f32),
          pltpu.VMEM_SHARED((NR, 128), _f32),
          pltpu.VMEM_SHARED((NR, 128), _f32),
          pltpu.VMEM_SHARED((NR, 128), _f32),
          pltpu.VMEM_SHARED((NS, CAP), _i32),
          pltpu.VMEM_SHARED((NS, CAP), _f32),
          pltpu.VMEM_SHARED((NPAD,), _f32),
      ],
  )(e0, e1, e2, e3, symx, xc1, xc2)


BR = 1024
GRID = NPAD // BR


def _k1_body(x_ref, wa_ref, wb_ref, wc_ref, oa_ref, ob_ref, oc_ref):
  xb = x_ref[...]
  oa_ref[...] = jnp.dot(xb, wa_ref[...], preferred_element_type=_f32)
  ob_ref[...] = jnp.dot(xb, wb_ref[...], preferred_element_type=_f32)
  oc_ref[...] = jnp.dot(xb, wc_ref[...], preferred_element_type=_f32)


def _k1(x_pad, wa, wb, wc):
  bs = pl.BlockSpec((BR, D), lambda i: (i, 0))
  ws = pl.BlockSpec((D, D), lambda i: (0, 0))
  return pl.pallas_call(
      _k1_body,
      grid=(GRID,),
      in_specs=[bs, ws, ws, ws],
      out_specs=[bs, bs, bs],
      out_shape=[jax.ShapeDtypeStruct((NPAD, D), _f32)] * 3,
      compiler_params=pltpu.CompilerParams(
          dimension_semantics=("arbitrary",)),
  )(x_pad, wa, wb, wc)


def _k2a_body(hacc_ref, x_ref, w_ref, badd_ref, h_ref, s1_ref, s2_ref):
  i = pl.program_id(0)
  hb = (hacc_ref[...] + badd_ref[...]
        + jnp.dot(x_ref[...], w_ref[...], preferred_element_type=_f32))
  row = i * BR + lax.broadcasted_iota(_i32, (BR, D), 0)
  hb = jnp.where(row < N, hb, 0.0)
  h_ref[...] = hb
  s1 = jnp.sum(hb, axis=0, keepdims=True)
  s2 = jnp.sum(hb * hb, axis=0, keepdims=True)
  r0 = lax.broadcasted_iota(_i32, (8, D), 0) == 0
  s1b = jnp.where(r0, jnp.broadcast_to(s1, (8, D)), 0.0)
  s2b = jnp.where(r0, jnp.broadcast_to(s2, (8, D)), 0.0)

  @pl.when(i == 0)
  def _():
    s1_ref[...] = s1b
    s2_ref[...] = s2b

  @pl.when(i > 0)
  def _():
    s1_ref[...] += s1b
    s2_ref[...] += s2b


def _k2a(hacc, x_pad, ln_w, badd):
  return pl.pallas_call(
      _k2a_body,
      grid=(GRID,),
      in_specs=[
          pl.BlockSpec((BR, D), lambda i: (i, 0)),
          pl.BlockSpec((BR, D), lambda i: (i, 0)),
          pl.BlockSpec((D, D), lambda i: (0, 0)),
          pl.BlockSpec((1, D), lambda i: (0, 0)),
      ],
      out_specs=[
          pl.BlockSpec((BR, D), lambda i: (i, 0)),
          pl.BlockSpec((8, D), lambda i: (0, 0)),
          pl.BlockSpec((8, D), lambda i: (0, 0)),
      ],
      out_shape=[
          jax.ShapeDtypeStruct((NPAD, D), _f32),
          jax.ShapeDtypeStruct((8, D), _f32),
          jax.ShapeDtypeStruct((8, D), _f32),
      ],
      compiler_params=pltpu.CompilerParams(
          dimension_semantics=("arbitrary",)),
  )(hacc, x_pad, ln_w, badd)


def _k2b_body(h_ref, s1_ref, s2_ref, cw_ref, cb_ref, g_ref, b_ref,
              h2_ref, t1_ref, t2_ref):
  i = pl.program_id(0)
  mu = s1_ref[0:1, :] * (1.0 / N)
  var = s2_ref[0:1, :] * (1.0 / N) - mu * mu
  sc = g_ref[...] * lax.rsqrt(var + EPS)
  hn = (h_ref[...] - mu) * sc + b_ref[...]
  h2 = jnp.dot(hn, cw_ref[...], preferred_element_type=_f32) + cb_ref[...]
  row = i * BR + lax.broadcasted_iota(_i32, (BR, OUT_DIM), 0)
  h2m = jnp.where(row < N, h2, 0.0)
  h2_ref[...] = h2
  t1 = jnp.sum(h2m, axis=0, keepdims=True)
  t2 = jnp.sum(h2m * h2m, axis=0, keepdims=True)
  r0 = lax.broadcasted_iota(_i32, (8, OUT_DIM), 0) == 0
  t1b = jnp.where(r0, jnp.broadcast_to(t1, (8, OUT_DIM)), 0.0)
  t2b = jnp.where(r0, jnp.broadcast_to(t2, (8, OUT_DIM)), 0.0)

  @pl.when(i == 0)
  def _():
    t1_ref[...] = t1b
    t2_ref[...] = t2b

  @pl.when(i > 0)
  def _():
    t1_ref[...] += t1b
    t2_ref[...] += t2b


def _k2b(h, s1, s2, conv_w, conv_b, bn1_g, bn1_b):
  return pl.pallas_call(
      _k2b_body,
      grid=(GRID,),
      in_specs=[
          pl.BlockSpec((BR, D), lambda i: (i, 0)),
          pl.BlockSpec((8, D), lambda i: (0, 0)),
          pl.BlockSpec((8, D), lambda i: (0, 0)),
          pl.BlockSpec((D, OUT_DIM), lambda i: (0, 0)),
          pl.BlockSpec((1, OUT_DIM), lambda i: (0, 0)),
          pl.BlockSpec((1, D), lambda i: (0, 0)),
          pl.BlockSpec((1, D), lambda i: (0, 0)),
      ],
      out_specs=[
          pl.BlockSpec((BR, OUT_DIM), lambda i: (i, 0)),
          pl.BlockSpec((8, OUT_DIM), lambda i: (0, 0)),
          pl.BlockSpec((8, OUT_DIM), lambda i: (0, 0)),
      ],
      out_shape=[
          jax.ShapeDtypeStruct((NPAD, OUT_DIM), _f32),
          jax.ShapeDtypeStruct((8, OUT_DIM), _f32),
          jax.ShapeDtypeStruct((8, OUT_DIM), _f32),
      ],
      compiler_params=pltpu.CompilerParams(
          dimension_semantics=("arbitrary",)),
  )(h, s1, s2, conv_w, conv_b, bn1_g, bn1_b)


def _k2c_body(h2_ref, t1_ref, t2_ref, g_ref, b_ref, o_ref):
  mu = t1_ref[0:1, :] * (1.0 / N)
  var = t2_ref[0:1, :] * (1.0 / N) - mu * mu
  o_ref[...] = ((h2_ref[...] - mu) * lax.rsqrt(var + EPS) * g_ref[...]
                + b_ref[...])


def _k2c(h2, t1, t2, bn2_g, bn2_b):
  return pl.pallas_call(
      _k2c_body,
      grid=(GRID,),
      in_specs=[
          pl.BlockSpec((BR, OUT_DIM), lambda i: (i, 0)),
          pl.BlockSpec((8, OUT_DIM), lambda i: (0, 0)),
          pl.BlockSpec((8, OUT_DIM), lambda i: (0, 0)),
          pl.BlockSpec((1, OUT_DIM), lambda i: (0, 0)),
          pl.BlockSpec((1, OUT_DIM), lambda i: (0, 0)),
      ],
      out_specs=pl.BlockSpec((BR, OUT_DIM), lambda i: (i, 0)),
      out_shape=jax.ShapeDtypeStruct((NPAD, OUT_DIM), _f32),
      compiler_params=pltpu.CompilerParams(
          dimension_semantics=("arbitrary",)),
  )(h2, t1, t2, bn2_g, bn2_b)


def kernel(x, edge_index, edge_in, in_w, edge_out, out_w, edge_index_tuple,
           edge_weight_tuple, lin1_w, ib1_ln_w, ib1_ln_b, ib1_c1_w, ib1_c1_b,
           ib1_c2_w, ib1_c2_b, conv_w, conv_b, bn1_g, bn1_b, bn2_g, bn2_b):
  def pack(u, v, w):
    return jnp.stack([u.astype(_i32), v.astype(_i32),
                      lax.bitcast_convert_type(w.astype(_f32), _i32)])

  e0 = pack(edge_index_tuple[0, 0], edge_index_tuple[0, 1],
            edge_weight_tuple[0])
  e1 = pack(edge_in[0], edge_in[1], in_w)
  e2 = pack(edge_out[0], edge_out[1], out_w)
  e3 = pack(edge_index_tuple[1, 0], edge_index_tuple[1, 1],
            edge_weight_tuple[1])

  x_pad = jnp.pad(x, ((0, NPAD - N), (0, 0)))
  symx, xc1, xc2 = _k1(x_pad, lin1_w, ib1_c1_w, ib1_c2_w)

  hacc = _sc_scatter(e0, e1, e2, e3, symx, xc1, xc2)

  badd = (ib1_ln_b + ib1_c1_b + ib1_c2_b).reshape(1, D)
  h, s1, s2 = _k2a(hacc, x_pad, ib1_ln_w, badd)
  h2, t1, t2 = _k2b(h, s1, s2, conv_w, conv_b.reshape(1, OUT_DIM),
                    bn1_g.reshape(1, D), bn1_b.reshape(1, D))
  out = _k2c(h2, t1, t2, bn2_g.reshape(1, OUT_DIM), bn2_b.reshape(1, OUT_DIM))
  return out[:N]
